# baseline (speedup 1.0000x reference)
import functools

import jax
import jax.numpy as jnp
from jax.experimental import pallas as pl
from jax.experimental.pallas import tpu as pltpu


def _round_up(x, m):
    return ((x + m - 1) // m) * m


def _fused_kernel(batch_ref, x_ref, w_ref, b_ref, o_ref, acc_ref, *,
                  n_nodes, nt):
    n = pl.program_id(0)
    tile_n = x_ref.shape[0]
    b_pad = acc_ref.shape[0]

    @pl.when(n == 0)
    def _():
        acc_ref[...] = jnp.zeros_like(acc_ref)

    x_blk = x_ref[...]
    if n_nodes % tile_n != 0:
        row = jax.lax.broadcasted_iota(jnp.int32, x_blk.shape, 0)
        x_blk = jnp.where(row + n * tile_n < n_nodes, x_blk, 0)

    gid = jax.lax.broadcasted_iota(jnp.int32, (b_pad, tile_n), 0)
    onehot = (gid == batch_ref[...]).astype(x_blk.dtype)

    acc_ref[...] += jax.lax.dot_general(
        onehot, x_blk,
        dimension_numbers=(((1,), (0,)), ((), ())),
        preferred_element_type=jnp.float32,
        precision=jax.lax.Precision.DEFAULT,
    )

    @pl.when(n == nt - 1)
    def _():
        out = jax.lax.dot_general(
            acc_ref[...], w_ref[...],
            dimension_numbers=(((1,), (1,)), ((), ())),
            preferred_element_type=jnp.float32,
            precision=jax.lax.Precision.HIGHEST,
        ) + b_ref[...]
        o_ref[...] = out.astype(o_ref.dtype)


@functools.partial(jax.jit, static_argnames=("num_graphs",))
def _decoder_forward(x, batch, weight, bias, num_graphs):
    N, H = x.shape
    B_pad = _round_up(max(num_graphs, 1), 8)

    TILE_N = 2048
    nt = pl.cdiv(N, TILE_N)
    N_pad = nt * TILE_N

    batch_p = jnp.full((1, N_pad), -1, dtype=jnp.int32).at[0, :N].set(
        batch.astype(jnp.int32))
    bias2d = bias.astype(jnp.float32).reshape(1, H)

    out_p = pl.pallas_call(
        functools.partial(_fused_kernel, n_nodes=N, nt=nt),
        out_shape=jax.ShapeDtypeStruct((B_pad, H), x.dtype),
        grid=(nt,),
        in_specs=[
            pl.BlockSpec((1, TILE_N), lambda n: (0, n)),
            pl.BlockSpec((TILE_N, H), lambda n: (n, 0)),
            pl.BlockSpec((H, H), lambda n: (0, 0)),
            pl.BlockSpec((1, H), lambda n: (0, 0)),
        ],
        out_specs=pl.BlockSpec((B_pad, H), lambda n: (0, 0)),
        scratch_shapes=[pltpu.VMEM((B_pad, H), jnp.float32)],
        compiler_params=pltpu.CompilerParams(
            dimension_semantics=("arbitrary",),
            vmem_limit_bytes=56 * 1024 * 1024,
        ),
    )(batch_p, x, weight, bias2d)

    return out_p[:num_graphs, :]


def kernel(x, batch, weight, bias):
    return _decoder_forward(x, batch, weight, bias, num_graphs=256)


# speedup vs baseline: 1.1914x; 1.1914x over previous
import functools

import jax
import jax.numpy as jnp
from jax.experimental import pallas as pl
from jax.experimental.pallas import tpu as pltpu


def _round_up(x, m):
    return ((x + m - 1) // m) * m


def _fused_kernel(batch_ref, x_ref, w_ref, b_ref, o_ref, acc_ref, *,
                  n_nodes, nt):
    n = pl.program_id(0)
    tile_n = x_ref.shape[0]
    b_pad = acc_ref.shape[0]

    @pl.when(n == 0)
    def _():
        acc_ref[...] = jnp.zeros_like(acc_ref)

    x_blk = x_ref[...]
    if n_nodes % tile_n != 0:
        row = jax.lax.broadcasted_iota(jnp.int32, x_blk.shape, 0)
        x_blk = jnp.where(row + n * tile_n < n_nodes, x_blk, 0)

    gid = jax.lax.broadcasted_iota(jnp.int32, (b_pad, tile_n), 0)
    onehot = (gid == batch_ref[...]).astype(x_blk.dtype)

    acc_ref[...] += jax.lax.dot_general(
        onehot, x_blk,
        dimension_numbers=(((1,), (0,)), ((), ())),
        preferred_element_type=jnp.float32,
        precision=jax.lax.Precision.DEFAULT,
    )

    @pl.when(n == nt - 1)
    def _():
        out = jax.lax.dot_general(
            acc_ref[...], w_ref[...],
            dimension_numbers=(((1,), (1,)), ((), ())),
            preferred_element_type=jnp.float32,
            precision=jax.lax.Precision.HIGHEST,
        ) + b_ref[...]
        o_ref[...] = out.astype(o_ref.dtype)


@functools.partial(jax.jit, static_argnames=("num_graphs",))
def _decoder_forward(x, batch, weight, bias, num_graphs):
    N, H = x.shape
    B_pad = _round_up(max(num_graphs, 1), 8)

    TILE_N = 4096
    nt = pl.cdiv(N, TILE_N)
    N_pad = nt * TILE_N

    batch_p = jnp.full((1, N_pad), -1, dtype=jnp.int32).at[0, :N].set(
        batch.astype(jnp.int32))
    bias2d = bias.astype(jnp.float32).reshape(1, H)

    out_p = pl.pallas_call(
        functools.partial(_fused_kernel, n_nodes=N, nt=nt),
        out_shape=jax.ShapeDtypeStruct((B_pad, H), x.dtype),
        grid=(nt,),
        in_specs=[
            pl.BlockSpec((1, TILE_N), lambda n: (0, n)),
            pl.BlockSpec((TILE_N, H), lambda n: (n, 0)),
            pl.BlockSpec((H, H), lambda n: (0, 0)),
            pl.BlockSpec((1, H), lambda n: (0, 0)),
        ],
        out_specs=pl.BlockSpec((B_pad, H), lambda n: (0, 0)),
        scratch_shapes=[pltpu.VMEM((B_pad, H), jnp.float32)],
        compiler_params=pltpu.CompilerParams(
            dimension_semantics=("arbitrary",),
            vmem_limit_bytes=56 * 1024 * 1024,
        ),
    )(batch_p, x, weight, bias2d)

    return out_p[:num_graphs, :]


def kernel(x, batch, weight, bias):
    return _decoder_forward(x, batch, weight, bias, num_graphs=256)


# speedup vs baseline: 1.1942x; 1.0024x over previous
import functools

import jax
import jax.numpy as jnp
from jax.experimental import pallas as pl
from jax.experimental.pallas import tpu as pltpu


def _round_up(x, m):
    return ((x + m - 1) // m) * m


def _fused_kernel(batch_ref, x_ref, w_ref, b_ref, o_ref, acc_ref, *,
                  n_nodes, nt):
    n = pl.program_id(0)
    tile_n = x_ref.shape[0]
    b_pad = acc_ref.shape[0]

    @pl.when(n == 0)
    def _():
        acc_ref[...] = jnp.zeros_like(acc_ref)

    x_blk = x_ref[...]
    if n_nodes % tile_n != 0:
        row = jax.lax.broadcasted_iota(jnp.int32, x_blk.shape, 0)
        x_blk = jnp.where(row + n * tile_n < n_nodes, x_blk, 0)

    gid = jax.lax.broadcasted_iota(jnp.int32, (b_pad, tile_n), 0)
    onehot = (gid == batch_ref[...]).astype(x_blk.dtype)

    acc_ref[...] += jax.lax.dot_general(
        onehot, x_blk,
        dimension_numbers=(((1,), (0,)), ((), ())),
        preferred_element_type=jnp.float32,
        precision=jax.lax.Precision.DEFAULT,
    )

    @pl.when(n == nt - 1)
    def _():
        out = jax.lax.dot_general(
            acc_ref[...], w_ref[...],
            dimension_numbers=(((1,), (1,)), ((), ())),
            preferred_element_type=jnp.float32,
            precision=jax.lax.Precision.HIGHEST,
        ) + b_ref[...]
        o_ref[...] = out.astype(o_ref.dtype)


@functools.partial(jax.jit, static_argnames=("num_graphs",))
def _decoder_forward(x, batch, weight, bias, num_graphs):
    N, H = x.shape
    B_pad = _round_up(max(num_graphs, 1), 8)

    TILE_N = 4096
    nt = pl.cdiv(N, TILE_N)
    N_pad = nt * TILE_N

    if N_pad == N:
        batch_p = batch.astype(jnp.int32).reshape(1, N)
    else:
        batch_p = jnp.full((1, N_pad), -1, dtype=jnp.int32).at[0, :N].set(
            batch.astype(jnp.int32))
    bias2d = bias.astype(jnp.float32).reshape(1, H)

    out_p = pl.pallas_call(
        functools.partial(_fused_kernel, n_nodes=N, nt=nt),
        out_shape=jax.ShapeDtypeStruct((B_pad, H), x.dtype),
        grid=(nt,),
        in_specs=[
            pl.BlockSpec((1, TILE_N), lambda n: (0, n)),
            pl.BlockSpec((TILE_N, H), lambda n: (n, 0)),
            pl.BlockSpec((H, H), lambda n: (0, 0)),
            pl.BlockSpec((1, H), lambda n: (0, 0)),
        ],
        out_specs=pl.BlockSpec((B_pad, H), lambda n: (0, 0)),
        scratch_shapes=[pltpu.VMEM((B_pad, H), jnp.float32)],
        compiler_params=pltpu.CompilerParams(
            dimension_semantics=("arbitrary",),
            vmem_limit_bytes=56 * 1024 * 1024,
        ),
    )(batch_p, x, weight, bias2d)

    return out_p[:num_graphs, :]


def kernel(x, batch, weight, bias):
    return _decoder_forward(x, batch, weight, bias, num_graphs=256)
